# baseline (speedup 1.0000x reference)
import functools

import jax
import jax.numpy as jnp
from jax import lax
from jax.experimental import pallas as pl
from jax.experimental.pallas import tpu as pltpu
from jax.experimental.pallas import tpu_sc as plsc

T, D, E, K = 4096, 768, 8, 2
BM = 128
NP = K * T + E * BM
NBLK = NP // BM
NC, NS = 2, 16
NW = NC * NS
PPW = (K * T) // NW
TPW = T // NW
CH = 128
CCH = 64

def _sc_mesh():
    return plsc.VectorSubcoreMesh(
        core_axis_name="c", subcore_axis_name="s",
        num_cores=NC, num_subcores=NS)


def _router_body(x_ref, wg_ref, bg_ref, pos_ref, gv_ref, blk_ref):
    xb = x_ref[...]
    logits = jnp.dot(xb, wg_ref[...], preferred_element_type=jnp.float32)
    logits = logits + bg_ref[...]
    m = jnp.max(logits, axis=-1, keepdims=True)
    p = jnp.exp(logits - m)
    p = p / jnp.sum(p, axis=-1, keepdims=True)

    iota = lax.broadcasted_iota(jnp.int32, (T, E), 1)
    v0 = jnp.max(p, axis=-1, keepdims=True)
    i0 = jnp.min(jnp.where(p >= v0, iota, E), axis=-1, keepdims=True)
    sel0 = iota == i0
    p2 = jnp.where(sel0, -jnp.inf, p)
    v1 = jnp.max(p2, axis=-1, keepdims=True)
    i1 = jnp.min(jnp.where(p2 >= v1, iota, E), axis=-1, keepdims=True)
    sel1 = iota == i1
    O0 = sel0.astype(jnp.float32)
    O1 = sel1.astype(jnp.float32)

    CB = 512
    r_i = lax.broadcasted_iota(jnp.int32, (CB, CB), 0)
    c_i = lax.broadcasted_iota(jnp.int32, (CB, CB), 1)
    Ls = (c_i < r_i).astype(jnp.float32)
    running = jnp.zeros((1, E), jnp.float32)
    rank_chunks = [[], []]
    for kk, O in enumerate((O0, O1)):
        for i in range(T // CB):
            ch = O[i * CB:(i + 1) * CB]
            rank_chunks[kk].append(
                jnp.dot(Ls, ch, preferred_element_type=jnp.float32,
                        precision=lax.Precision.HIGHEST) + running)
            running = running + jnp.sum(ch, axis=0, keepdims=True)
    counts = running

    nb = (counts.astype(jnp.int32) + (BM - 1)) // BM
    nbf = nb.astype(jnp.float32)
    e_r = lax.broadcasted_iota(jnp.int32, (E, E), 0)
    e_c = lax.broadcasted_iota(jnp.int32, (E, E), 1)
    Le = (e_r < e_c).astype(jnp.float32)
    bstart = jnp.dot(nbf, Le, preferred_element_type=jnp.float32,
                     precision=lax.Precision.HIGHEST)
    base = bstart * BM

    I512 = (r_i == c_i).astype(jnp.float32)
    zpad = jnp.zeros((CB, E - 2), jnp.float32)
    dn = (((0,), (0,)), ((), ()))
    for i in range(T // CB):
        sl = slice(i * CB, (i + 1) * CB)
        p0c = jnp.sum(O0[sl] * (rank_chunks[0][i] + base), axis=1,
                      keepdims=True)
        p1c = jnp.sum(O1[sl] * (rank_chunks[1][i] + base), axis=1,
                      keepdims=True)
        Mp = jnp.concatenate([p0c, p1c, zpad], axis=1)
        Mg = jnp.concatenate([v0[sl], v1[sl], zpad], axis=1)
        pT = lax.dot_general(Mp, I512, dn, preferred_element_type=jnp.float32,
                             precision=lax.Precision.HIGHEST)
        gT = lax.dot_general(Mg, I512, dn, preferred_element_type=jnp.float32,
                             precision=lax.Precision.HIGHEST)
        pos_ref[:, sl] = pT.astype(jnp.int32)
        gv_ref[:, sl] = gT

    bend = bstart + nbf
    bi = lax.broadcasted_iota(jnp.int32, (NBLK, E), 0).astype(jnp.float32)
    eidx = jnp.sum((bi >= bend).astype(jnp.float32), axis=1, keepdims=True)
    blk_ref[:, 0:1] = jnp.minimum(eidx, E - 1).astype(jnp.int32)


def _router(x, Wg, bg):
    return pl.pallas_call(
        _router_body,
        out_shape=[
            jax.ShapeDtypeStruct((8, T), jnp.int32),
            jax.ShapeDtypeStruct((8, T), jnp.float32),
            jax.ShapeDtypeStruct((NBLK, 8), jnp.int32),
        ],
    )(x, Wg, bg.reshape(1, E))


def _dispatch_body(x_hbm, pos_hbm, xs_hbm, pidx, xbuf, sem):
    wid = lax.axis_index("s") * NC + lax.axis_index("c")
    for j in range(PPW // CH):
        pbase = wid * PPW + j * CH
        k = pbase // T
        t0 = pbase - k * T
        pltpu.sync_copy(pos_hbm.at[k, pl.ds(t0, CH)], pidx)
        pltpu.sync_copy(x_hbm.at[pl.ds(t0, CH), :], xbuf)
        pltpu.async_copy(xbuf, xs_hbm.at[pidx], sem).wait()


def _gemm_body(blk_ref, xs_ref, we_ref, be_ref, ys_ref):
    ys_ref[...] = jnp.dot(xs_ref[...], we_ref[0],
                          preferred_element_type=jnp.float32) + be_ref[0]


def _gemm(blk1d, xs, We, be):
    grid_spec = pltpu.PrefetchScalarGridSpec(
        num_scalar_prefetch=1,
        grid=(NBLK,),
        in_specs=[
            pl.BlockSpec((BM, D), lambda i, blk: (i, 0)),
            pl.BlockSpec((1, D, D), lambda i, blk: (blk[i], 0, 0)),
            pl.BlockSpec((1, 1, D), lambda i, blk: (blk[i], 0, 0)),
        ],
        out_specs=pl.BlockSpec((BM, D), lambda i, blk: (i, 0)),
    )
    return pl.pallas_call(
        _gemm_body,
        grid_spec=grid_spec,
        out_shape=jax.ShapeDtypeStruct((NP, D), jnp.float32),
    )(blk1d, xs, We, be.reshape(E, 1, D))


def _combine_body(ys_hbm, pos_hbm, gv_hbm, out_hbm,
                  g0b, g1b, pidx0, pidx1, abuf, bbuf, sem):
    wid = lax.axis_index("s") * NC + lax.axis_index("c")
    i16 = lax.iota(jnp.int32, 16)
    zf = jnp.zeros((16,), jnp.float32)
    t0 = wid * TPW
    pltpu.sync_copy(gv_hbm.at[0, pl.ds(t0, TPW)], g0b)
    pltpu.sync_copy(gv_hbm.at[1, pl.ds(t0, TPW)], g1b)
    for j in range(TPW // CCH):
        pltpu.sync_copy(pos_hbm.at[0, pl.ds(t0 + j * CCH, CCH)], pidx0)
        pltpu.sync_copy(pos_hbm.at[1, pl.ds(t0 + j * CCH, CCH)], pidx1)
        ca = pltpu.async_copy(ys_hbm.at[pidx0], abuf, sem)
        cb = pltpu.async_copy(ys_hbm.at[pidx1], bbuf, sem)
        ca.wait()
        cb.wait()

        def grp_body(m, _):
            gv16_0 = g0b[pl.ds(j * CCH + m * 16, 16)]
            gv16_1 = g1b[pl.ds(j * CCH + m * 16, 16)]
            for r in range(16):
                g0 = zf + gv16_0[r]
                g1 = zf + gv16_1[r]
                i = m * 16 + r

                def col_body(cc, _, i=i, g0=g0, g1=g1):
                    a = abuf[i, pl.ds(cc * 16, 16)]
                    b = bbuf[i, pl.ds(cc * 16, 16)]
                    abuf[i, pl.ds(cc * 16, 16)] = a * g0 + b * g1
                    return 0

                lax.fori_loop(0, D // 16, col_body, 0)
            return 0

        lax.fori_loop(0, CCH // 16, grp_body, 0)
        pltpu.sync_copy(abuf, out_hbm.at[pl.ds(t0 + j * CCH, CCH), :])


def kernel(x, Wg, bg, We, be):
    dispatch = pl.kernel(
        _dispatch_body,
        out_type=jax.ShapeDtypeStruct((NP, D), jnp.float32),
        mesh=_sc_mesh(),
        scratch_types=[
            pltpu.VMEM((CH,), jnp.int32),
            pltpu.VMEM((CH, D), jnp.float32),
            pltpu.SemaphoreType.DMA,
        ],
    )
    combine = pl.kernel(
        _combine_body,
        out_type=jax.ShapeDtypeStruct((T, D), jnp.float32),
        mesh=_sc_mesh(),
        scratch_types=[
            pltpu.VMEM((TPW,), jnp.float32),
            pltpu.VMEM((TPW,), jnp.float32),
            pltpu.VMEM((CCH,), jnp.int32),
            pltpu.VMEM((CCH,), jnp.int32),
            pltpu.VMEM((CCH, D), jnp.float32),
            pltpu.VMEM((CCH, D), jnp.float32),
            pltpu.SemaphoreType.DMA,
        ],
    )
    posT, gvT, blk8 = _router(x, Wg, bg)
    xs = dispatch(x, posT)
    ys = _gemm(blk8[:, 0], xs, We, be)
    return combine(ys, posT, gvT)


# speedup vs baseline: 1.0157x; 1.0157x over previous
import jax
import jax.numpy as jnp
from jax import lax
from jax.experimental import pallas as pl
from jax.experimental.pallas import tpu as pltpu
from jax.experimental.pallas import tpu_sc as plsc

T, D, E, K = 4096, 768, 8, 2
NG = E * E
NPAIR = (E * (E - 1)) // 2
BM = 64
NP = T + NPAIR * BM
NBLK = NP // BM
NC, NS = 2, 16
NW = NC * NS
TPW = T // NW

def _sc_mesh():
    return plsc.VectorSubcoreMesh(
        core_axis_name="c", subcore_axis_name="s",
        num_cores=NC, num_subcores=NS)


def _router_body(x_ref, wg_ref, bg_ref, pos_ref, gv_ref, blk_ref):
    xb = x_ref[...]
    logits = jnp.dot(xb, wg_ref[...], preferred_element_type=jnp.float32)
    logits = logits + bg_ref[...]
    m = jnp.max(logits, axis=-1, keepdims=True)
    p = jnp.exp(logits - m)
    p = p / jnp.sum(p, axis=-1, keepdims=True)

    iota = lax.broadcasted_iota(jnp.int32, (T, E), 1)
    v0 = jnp.max(p, axis=-1, keepdims=True)
    i0 = jnp.min(jnp.where(p >= v0, iota, E), axis=-1, keepdims=True)
    sel0 = iota == i0
    p2 = jnp.where(sel0, -jnp.inf, p)
    v1 = jnp.max(p2, axis=-1, keepdims=True)
    i1 = jnp.min(jnp.where(p2 >= v1, iota, E), axis=-1, keepdims=True)

    emin = jnp.minimum(i0, i1)
    emax = jnp.maximum(i0, i1)
    ga = jnp.where(i0 < i1, v0, v1)
    gb = jnp.where(i0 < i1, v1, v0)
    gid = emin * E + emax
    iota_g = lax.broadcasted_iota(jnp.int32, (T, NG), 1)
    O = (iota_g == gid).astype(jnp.float32)

    CB = 512
    r_i = lax.broadcasted_iota(jnp.int32, (CB, CB), 0)
    c_i = lax.broadcasted_iota(jnp.int32, (CB, CB), 1)
    Ls = (c_i < r_i).astype(jnp.float32)
    running = jnp.zeros((1, NG), jnp.float32)
    rank_chunks = []
    for i in range(T // CB):
        ch = O[i * CB:(i + 1) * CB]
        rank_chunks.append(
            jnp.dot(Ls, ch, preferred_element_type=jnp.float32,
                    precision=lax.Precision.HIGHEST) + running)
        running = running + jnp.sum(ch, axis=0, keepdims=True)
    counts = running

    nb = (counts.astype(jnp.int32) + (BM - 1)) // BM
    nbf = nb.astype(jnp.float32)
    g_r = lax.broadcasted_iota(jnp.int32, (NG, NG), 0)
    g_c = lax.broadcasted_iota(jnp.int32, (NG, NG), 1)
    Lg = (g_r < g_c).astype(jnp.float32)
    bstart = jnp.dot(nbf, Lg, preferred_element_type=jnp.float32,
                     precision=lax.Precision.HIGHEST)
    base = bstart * BM

    I512 = (r_i == c_i).astype(jnp.float32)
    zpad7 = jnp.zeros((CB, 7), jnp.float32)
    zpad6 = jnp.zeros((CB, 6), jnp.float32)
    dn = (((0,), (0,)), ((), ()))
    for i in range(T // CB):
        sl = slice(i * CB, (i + 1) * CB)
        pc = jnp.sum(O[sl] * (rank_chunks[i] + base), axis=1, keepdims=True)
        Mp = jnp.concatenate([pc, zpad7], axis=1)
        Mg = jnp.concatenate([ga[sl], gb[sl], zpad6], axis=1)
        pT = lax.dot_general(Mp, I512, dn, preferred_element_type=jnp.float32,
                             precision=lax.Precision.HIGHEST)
        gT = lax.dot_general(Mg, I512, dn, preferred_element_type=jnp.float32,
                             precision=lax.Precision.HIGHEST)
        pos_ref[:, sl] = pT.astype(jnp.int32)
        gv_ref[:, sl] = gT

    bend = bstart + nbf
    bi = lax.broadcasted_iota(jnp.int32, (NBLK, NG), 0).astype(jnp.float32)
    gidx = jnp.sum((bi >= bend).astype(jnp.float32), axis=1, keepdims=True)
    gidx = jnp.minimum(gidx, NG - 1).astype(jnp.int32)
    blk_ref[:, 0:1] = gidx // E
    blk_ref[:, 1:2] = gidx - (gidx // E) * E


def _router(x, Wg, bg):
    return pl.pallas_call(
        _router_body,
        out_shape=[
            jax.ShapeDtypeStruct((8, T), jnp.int32),
            jax.ShapeDtypeStruct((8, T), jnp.float32),
            jax.ShapeDtypeStruct((NBLK, 8), jnp.int32),
        ],
    )(x, Wg, bg.reshape(1, E))


def _dispatch_body(x_hbm, pos_hbm, gv_hbm, xs_hbm, gas_hbm, gbs_hbm,
                   pidx, xbuf, gab, gbb, sem):
    wid = lax.axis_index("s") * NC + lax.axis_index("c")
    t0 = wid * TPW
    pltpu.sync_copy(pos_hbm.at[0, pl.ds(t0, TPW)], pidx)
    pltpu.sync_copy(x_hbm.at[pl.ds(t0, TPW), :], xbuf)
    pltpu.sync_copy(gv_hbm.at[0, pl.ds(t0, TPW)], gab)
    pltpu.sync_copy(gv_hbm.at[1, pl.ds(t0, TPW)], gbb)
    ca = pltpu.async_copy(xbuf, xs_hbm.at[pidx], sem)
    cb = pltpu.async_copy(gab, gas_hbm.at[pidx], sem)
    cc = pltpu.async_copy(gbb, gbs_hbm.at[pidx], sem)
    ca.wait()
    cb.wait()
    cc.wait()


def _gemm_body(blka_ref, blkb_ref, xs_ref, we_ref, be_ref, ga_ref, gb_ref,
               ys_ref):
    i = pl.program_id(0)
    a = blka_ref[i]
    b = blkb_ref[i]
    xsb = xs_ref[...]
    ya = jnp.dot(xsb, we_ref[a], preferred_element_type=jnp.float32)
    yb = jnp.dot(xsb, we_ref[b], preferred_element_type=jnp.float32)
    ys_ref[...] = (ga_ref[...] * (ya + be_ref[a])
                   + gb_ref[...] * (yb + be_ref[b]))


def _gemm(blka, blkb, xs, We, be, gas, gbs):
    grid_spec = pltpu.PrefetchScalarGridSpec(
        num_scalar_prefetch=2,
        grid=(NBLK,),
        in_specs=[
            pl.BlockSpec((BM, D), lambda i, a, b: (i, 0)),
            pl.BlockSpec((E, D, D), lambda i, a, b: (0, 0, 0)),
            pl.BlockSpec((E, D), lambda i, a, b: (0, 0)),
            pl.BlockSpec((BM, 1), lambda i, a, b: (i, 0)),
            pl.BlockSpec((BM, 1), lambda i, a, b: (i, 0)),
        ],
        out_specs=pl.BlockSpec((BM, D), lambda i, a, b: (i, 0)),
    )
    return pl.pallas_call(
        _gemm_body,
        grid_spec=grid_spec,
        out_shape=jax.ShapeDtypeStruct((NP, D), jnp.float32),
    )(blka, blkb, xs, We, be, gas.reshape(NP, 1), gbs.reshape(NP, 1))


def _combine_body(ys_hbm, pos_hbm, out_hbm, pidx, buf, sem):
    wid = lax.axis_index("s") * NC + lax.axis_index("c")
    t0 = wid * TPW
    pltpu.sync_copy(pos_hbm.at[0, pl.ds(t0, TPW)], pidx)
    pltpu.async_copy(ys_hbm.at[pidx], buf, sem).wait()
    pltpu.sync_copy(buf, out_hbm.at[pl.ds(t0, TPW), :])


def kernel(x, Wg, bg, We, be):
    dispatch = pl.kernel(
        _dispatch_body,
        out_type=[
            jax.ShapeDtypeStruct((NP, D), jnp.float32),
            jax.ShapeDtypeStruct((NP,), jnp.float32),
            jax.ShapeDtypeStruct((NP,), jnp.float32),
        ],
        mesh=_sc_mesh(),
        scratch_types=[
            pltpu.VMEM((TPW,), jnp.int32),
            pltpu.VMEM((TPW, D), jnp.float32),
            pltpu.VMEM((TPW,), jnp.float32),
            pltpu.VMEM((TPW,), jnp.float32),
            pltpu.SemaphoreType.DMA,
        ],
    )
    combine = pl.kernel(
        _combine_body,
        out_type=jax.ShapeDtypeStruct((T, D), jnp.float32),
        mesh=_sc_mesh(),
        scratch_types=[
            pltpu.VMEM((TPW,), jnp.int32),
            pltpu.VMEM((TPW, D), jnp.float32),
            pltpu.SemaphoreType.DMA,
        ],
    )
    posT, gvT, blk8 = _router(x, Wg, bg)
    xs, gas, gbs = dispatch(x, posT, gvT)
    ys = _gemm(blk8[:, 0], blk8[:, 1], xs, We, be, gas, gbs)
    return combine(ys, posT)


# speedup vs baseline: 1.2692x; 1.2496x over previous
import jax
import jax.numpy as jnp
from jax import lax
from jax.experimental import pallas as pl
from jax.experimental.pallas import tpu as pltpu
from jax.experimental.pallas import tpu_sc as plsc

T, D, E, K = 4096, 768, 8, 2
NG = E * E
NPAIR = (E * (E - 1)) // 2
BM = 64
NP = T + NPAIR * BM
NBLK = NP // BM
NC, NS = 2, 16
NW = NC * NS
TPW = T // NW

def _sc_mesh():
    return plsc.VectorSubcoreMesh(
        core_axis_name="c", subcore_axis_name="s",
        num_cores=NC, num_subcores=NS)


def _router_body(x_ref, wg_ref, bg_ref, pos_ref, gv_ref, blk_ref):
    xb = x_ref[...]
    logits = jnp.dot(xb, wg_ref[...], preferred_element_type=jnp.float32)
    logits = logits + bg_ref[...]
    m = jnp.max(logits, axis=-1, keepdims=True)
    p = jnp.exp(logits - m)
    p = p / jnp.sum(p, axis=-1, keepdims=True)

    iota = lax.broadcasted_iota(jnp.int32, (T, E), 1)
    v0 = jnp.max(p, axis=-1, keepdims=True)
    i0 = jnp.min(jnp.where(p >= v0, iota, E), axis=-1, keepdims=True)
    sel0 = iota == i0
    p2 = jnp.where(sel0, -jnp.inf, p)
    v1 = jnp.max(p2, axis=-1, keepdims=True)
    i1 = jnp.min(jnp.where(p2 >= v1, iota, E), axis=-1, keepdims=True)

    emin = jnp.minimum(i0, i1)
    emax = jnp.maximum(i0, i1)
    ga = jnp.where(i0 < i1, v0, v1)
    gb = jnp.where(i0 < i1, v1, v0)
    gid = emin * E + emax
    iota_g = lax.broadcasted_iota(jnp.int32, (T, NG), 1)
    O = (iota_g == gid).astype(jnp.float32)

    CB = 512
    r_i = lax.broadcasted_iota(jnp.int32, (CB, CB), 0)
    c_i = lax.broadcasted_iota(jnp.int32, (CB, CB), 1)
    Ls = (c_i < r_i).astype(jnp.float32)
    running = jnp.zeros((1, NG), jnp.float32)
    rank_chunks = []
    for i in range(T // CB):
        ch = O[i * CB:(i + 1) * CB]
        rank_chunks.append(
            jnp.dot(Ls, ch, preferred_element_type=jnp.float32,
                    precision=lax.Precision.HIGHEST) + running)
        running = running + jnp.sum(ch, axis=0, keepdims=True)
    counts = running

    nb = (counts.astype(jnp.int32) + (BM - 1)) // BM
    nbf = nb.astype(jnp.float32)
    g_r = lax.broadcasted_iota(jnp.int32, (NG, NG), 0)
    g_c = lax.broadcasted_iota(jnp.int32, (NG, NG), 1)
    Lg = (g_r < g_c).astype(jnp.float32)
    bstart = jnp.dot(nbf, Lg, preferred_element_type=jnp.float32,
                     precision=lax.Precision.HIGHEST)
    base = bstart * BM

    I512 = (r_i == c_i).astype(jnp.float32)
    zpad7 = jnp.zeros((CB, 7), jnp.float32)
    zpad6 = jnp.zeros((CB, 6), jnp.float32)
    dn = (((0,), (0,)), ((), ()))
    for i in range(T // CB):
        sl = slice(i * CB, (i + 1) * CB)
        pc = jnp.sum(O[sl] * (rank_chunks[i] + base), axis=1, keepdims=True)
        Mp = jnp.concatenate([pc, zpad7], axis=1)
        Mg = jnp.concatenate([ga[sl], gb[sl], zpad6], axis=1)
        pT = lax.dot_general(Mp, I512, dn, preferred_element_type=jnp.float32,
                             precision=lax.Precision.HIGHEST)
        gT = lax.dot_general(Mg, I512, dn, preferred_element_type=jnp.float32,
                             precision=lax.Precision.HIGHEST)
        pos_ref[:, sl] = pT.astype(jnp.int32)
        gv_ref[:, sl] = gT

    bend = bstart + nbf
    bi = lax.broadcasted_iota(jnp.int32, (NBLK, NG), 0).astype(jnp.float32)
    gidx = jnp.sum((bi >= bend).astype(jnp.float32), axis=1, keepdims=True)
    gidx = jnp.minimum(gidx, NG - 1).astype(jnp.int32)
    blk_ref[:, 0:1] = gidx // E
    blk_ref[:, 1:2] = gidx - (gidx // E) * E


def _router(x, Wg, bg):
    return pl.pallas_call(
        _router_body,
        out_shape=[
            jax.ShapeDtypeStruct((8, T), jnp.int32),
            jax.ShapeDtypeStruct((8, T), jnp.float32),
            jax.ShapeDtypeStruct((NBLK, 8), jnp.int32),
        ],
    )(x, Wg, bg.reshape(1, E))


def _dispatch_body(x_hbm, pos_hbm, gv_hbm, xs_hbm, gs_hbm,
                   pidx, xbuf, gab, gbb, grows, sem):
    wid = lax.axis_index("s") * NC + lax.axis_index("c")
    i16 = lax.iota(jnp.int32, 16)
    zf = jnp.zeros((16,), jnp.float32)
    t0 = wid * TPW
    pltpu.sync_copy(pos_hbm.at[0, pl.ds(t0, TPW)], pidx)
    pltpu.sync_copy(x_hbm.at[pl.ds(t0, TPW), :], xbuf)
    pltpu.sync_copy(gv_hbm.at[0, pl.ds(t0, TPW)], gab)
    pltpu.sync_copy(gv_hbm.at[1, pl.ds(t0, TPW)], gbb)
    ca = pltpu.async_copy(xbuf, xs_hbm.at[pidx], sem)

    def grp_body(m, _):
        ga16 = gab[pl.ds(m * 16, 16)]
        gb16 = gbb[pl.ds(m * 16, 16)]
        for r in range(16):
            v = jnp.where(i16 == 0, zf + ga16[r],
                          jnp.where(i16 == 1, zf + gb16[r], 0.0))
            grows[m * 16 + r, pl.ds(0, 16)] = v
        return 0

    lax.fori_loop(0, TPW // 16, grp_body, 0)
    cb = pltpu.async_copy(grows, gs_hbm.at[pidx], sem)
    ca.wait()
    cb.wait()


def _gemm_body(blk_ref, xs_ref, we_ref, be_ref, gs_ref, ys_ref):
    i = pl.program_id(0)
    a = blk_ref[i, 0]
    b = blk_ref[i, 1]
    xsb = xs_ref[...]
    ya = jnp.dot(xsb, we_ref[a], preferred_element_type=jnp.float32)
    yb = jnp.dot(xsb, we_ref[b], preferred_element_type=jnp.float32)
    ys_ref[...] = (gs_ref[:, 0:1] * (ya + be_ref[a])
                   + gs_ref[:, 1:2] * (yb + be_ref[b]))


def _gemm(blk8, xs, We, be, gs):
    grid_spec = pltpu.PrefetchScalarGridSpec(
        num_scalar_prefetch=1,
        grid=(NBLK,),
        in_specs=[
            pl.BlockSpec((BM, D), lambda i, blk: (i, 0)),
            pl.BlockSpec((E, D, D), lambda i, blk: (0, 0, 0)),
            pl.BlockSpec((E, D), lambda i, blk: (0, 0)),
            pl.BlockSpec((BM, 128), lambda i, blk: (i, 0)),
        ],
        out_specs=pl.BlockSpec((BM, D), lambda i, blk: (i, 0)),
    )
    return pl.pallas_call(
        _gemm_body,
        grid_spec=grid_spec,
        out_shape=jax.ShapeDtypeStruct((NP, D), jnp.float32),
    )(blk8, xs, We, be, gs)


def _combine_body(ys_hbm, pos_hbm, out_hbm, pidx, buf, sem):
    wid = lax.axis_index("s") * NC + lax.axis_index("c")
    t0 = wid * TPW
    pltpu.sync_copy(pos_hbm.at[0, pl.ds(t0, TPW)], pidx)
    pltpu.async_copy(ys_hbm.at[pidx], buf, sem).wait()
    pltpu.sync_copy(buf, out_hbm.at[pl.ds(t0, TPW), :])


def kernel(x, Wg, bg, We, be):
    dispatch = pl.kernel(
        _dispatch_body,
        out_type=[
            jax.ShapeDtypeStruct((NP, D), jnp.float32),
            jax.ShapeDtypeStruct((NP, 128), jnp.float32),
        ],
        mesh=_sc_mesh(),
        scratch_types=[
            pltpu.VMEM((TPW,), jnp.int32),
            pltpu.VMEM((TPW, D), jnp.float32),
            pltpu.VMEM((TPW,), jnp.float32),
            pltpu.VMEM((TPW,), jnp.float32),
            pltpu.VMEM((TPW, 128), jnp.float32),
            pltpu.SemaphoreType.DMA,
        ],
    )
    combine = pl.kernel(
        _combine_body,
        out_type=jax.ShapeDtypeStruct((T, D), jnp.float32),
        mesh=_sc_mesh(),
        scratch_types=[
            pltpu.VMEM((TPW,), jnp.int32),
            pltpu.VMEM((TPW, D), jnp.float32),
            pltpu.SemaphoreType.DMA,
        ],
    )
    posT, gvT, blk8 = _router(x, Wg, bg)
    xs, gs = dispatch(x, posT, gvT)
    ys = _gemm(blk8, xs, We, be, gs)
    return combine(ys, posT)


# speedup vs baseline: 1.3332x; 1.0504x over previous
import jax
import jax.numpy as jnp
from jax import lax
from jax.experimental import pallas as pl
from jax.experimental.pallas import tpu as pltpu
from jax.experimental.pallas import tpu_sc as plsc

T, D, E, K = 4096, 768, 8, 2
NG = E * E
NPAIR = (E * (E - 1)) // 2
BM = 64
NP = T + NPAIR * BM
NBLK = NP // BM
NC, NS = 2, 16
NW = NC * NS
TPW = T // NW

def _sc_mesh():
    return plsc.VectorSubcoreMesh(
        core_axis_name="c", subcore_axis_name="s",
        num_cores=NC, num_subcores=NS)


def _router_body(x_ref, wg_ref, bg_ref, pos_ref, gv_ref, blk_ref):
    xb = x_ref[...]
    logits = jnp.dot(xb, wg_ref[...], preferred_element_type=jnp.float32)
    logits = logits + bg_ref[...]
    m = jnp.max(logits, axis=-1, keepdims=True)
    p = jnp.exp(logits - m)
    p = p / jnp.sum(p, axis=-1, keepdims=True)

    iota = lax.broadcasted_iota(jnp.int32, (T, E), 1)
    v0 = jnp.max(p, axis=-1, keepdims=True)
    i0 = jnp.min(jnp.where(p >= v0, iota, E), axis=-1, keepdims=True)
    sel0 = iota == i0
    p2 = jnp.where(sel0, -jnp.inf, p)
    v1 = jnp.max(p2, axis=-1, keepdims=True)
    i1 = jnp.min(jnp.where(p2 >= v1, iota, E), axis=-1, keepdims=True)

    emin = jnp.minimum(i0, i1)
    emax = jnp.maximum(i0, i1)
    ga = jnp.where(i0 < i1, v0, v1)
    gb = jnp.where(i0 < i1, v1, v0)
    gid = emin * E + emax
    iota_g = lax.broadcasted_iota(jnp.int32, (T, NG), 1)
    O = (iota_g == gid).astype(jnp.float32)

    CB = 512
    r_i = lax.broadcasted_iota(jnp.int32, (CB, CB), 0)
    c_i = lax.broadcasted_iota(jnp.int32, (CB, CB), 1)
    Ls = (c_i < r_i).astype(jnp.float32)
    running = jnp.zeros((1, NG), jnp.float32)
    rank_chunks = []
    for i in range(T // CB):
        ch = O[i * CB:(i + 1) * CB]
        rank_chunks.append(
            jnp.dot(Ls, ch, preferred_element_type=jnp.float32) + running)
        running = running + jnp.sum(ch, axis=0, keepdims=True)
    counts = running

    nb = (counts.astype(jnp.int32) + (BM - 1)) // BM
    nbf = nb.astype(jnp.float32)
    g_r = lax.broadcasted_iota(jnp.int32, (NG, NG), 0)
    g_c = lax.broadcasted_iota(jnp.int32, (NG, NG), 1)
    Lg = (g_r < g_c).astype(jnp.float32)
    bstart = jnp.dot(nbf, Lg, preferred_element_type=jnp.float32)
    base = bstart * BM

    I512 = (r_i == c_i).astype(jnp.float32)
    zpad7 = jnp.zeros((CB, 7), jnp.float32)
    zpad6 = jnp.zeros((CB, 6), jnp.float32)
    dn = (((0,), (0,)), ((), ()))
    for i in range(T // CB):
        sl = slice(i * CB, (i + 1) * CB)
        pc = jnp.sum(O[sl] * (rank_chunks[i] + base), axis=1, keepdims=True)
        Mp = jnp.concatenate([pc, zpad7], axis=1)
        Mg = jnp.concatenate([ga[sl], gb[sl], zpad6], axis=1)
        pT = lax.dot_general(Mp, I512, dn, preferred_element_type=jnp.float32,
                             precision=lax.Precision.HIGHEST)
        gT = lax.dot_general(Mg, I512, dn, preferred_element_type=jnp.float32,
                             precision=lax.Precision.HIGHEST)
        pos_ref[:, sl] = pT.astype(jnp.int32)
        gv_ref[:, sl] = gT

    bend = bstart + nbf
    bi = lax.broadcasted_iota(jnp.int32, (NBLK, NG), 0).astype(jnp.float32)
    gidx = jnp.sum((bi >= bend).astype(jnp.float32), axis=1, keepdims=True)
    gidx = jnp.minimum(gidx, NG - 1).astype(jnp.int32)
    blk_ref[:, 0:1] = gidx // E
    blk_ref[:, 1:2] = gidx - (gidx // E) * E


def _router(x, Wg, bg):
    return pl.pallas_call(
        _router_body,
        out_shape=[
            jax.ShapeDtypeStruct((8, T), jnp.int32),
            jax.ShapeDtypeStruct((8, T), jnp.float32),
            jax.ShapeDtypeStruct((NBLK, 8), jnp.int32),
        ],
    )(x, Wg, bg.reshape(1, E))


def _dispatch_body(x_hbm, pos_hbm, gv_hbm, xs_hbm, gs_hbm,
                   pidx, xbuf, gab, gbb, grows, sem):
    wid = lax.axis_index("s") * NC + lax.axis_index("c")
    i16 = lax.iota(jnp.int32, 16)
    zf = jnp.zeros((16,), jnp.float32)
    t0 = wid * TPW
    pltpu.sync_copy(pos_hbm.at[0, pl.ds(t0, TPW)], pidx)
    pltpu.sync_copy(x_hbm.at[pl.ds(t0, TPW), :], xbuf)
    pltpu.sync_copy(gv_hbm.at[0, pl.ds(t0, TPW)], gab)
    pltpu.sync_copy(gv_hbm.at[1, pl.ds(t0, TPW)], gbb)
    ca = pltpu.async_copy(xbuf, xs_hbm.at[pidx], sem)

    def grp_body(m, _):
        ga16 = gab[pl.ds(m * 16, 16)]
        gb16 = gbb[pl.ds(m * 16, 16)]
        for r in range(16):
            v = jnp.where(i16 == 0, zf + ga16[r],
                          jnp.where(i16 == 1, zf + gb16[r], 0.0))
            grows[m * 16 + r, pl.ds(0, 16)] = v
        return 0

    lax.fori_loop(0, TPW // 16, grp_body, 0)
    cb = pltpu.async_copy(grows, gs_hbm.at[pidx], sem)
    ca.wait()
    cb.wait()


def _gemm_body(blk_ref, xs_ref, we_ref, be_ref, gs_ref, ys_ref, webf):
    i = pl.program_id(0)

    @pl.when(i == 0)
    def _():
        webf[...] = we_ref[...].astype(jnp.bfloat16)

    a = blk_ref[i, 0]
    b = blk_ref[i, 1]
    xsb = xs_ref[...].astype(jnp.bfloat16)
    ya = jnp.dot(xsb, webf[a], preferred_element_type=jnp.float32)
    yb = jnp.dot(xsb, webf[b], preferred_element_type=jnp.float32)
    ys_ref[...] = (gs_ref[:, 0:1] * (ya + be_ref[a])
                   + gs_ref[:, 1:2] * (yb + be_ref[b]))


def _gemm(blk8, xs, We, be, gs):
    grid_spec = pltpu.PrefetchScalarGridSpec(
        num_scalar_prefetch=1,
        grid=(NBLK,),
        in_specs=[
            pl.BlockSpec((BM, D), lambda i, blk: (i, 0)),
            pl.BlockSpec((E, D, D), lambda i, blk: (0, 0, 0)),
            pl.BlockSpec((E, D), lambda i, blk: (0, 0)),
            pl.BlockSpec((BM, 128), lambda i, blk: (i, 0)),
        ],
        out_specs=pl.BlockSpec((BM, D), lambda i, blk: (i, 0)),
        scratch_shapes=[pltpu.VMEM((E, D, D), jnp.bfloat16)],
    )
    return pl.pallas_call(
        _gemm_body,
        grid_spec=grid_spec,
        out_shape=jax.ShapeDtypeStruct((NP, D), jnp.float32),
    )(blk8, xs, We, be, gs)


def _combine_body(ys_hbm, pos_hbm, out_hbm, pidx, buf, sem):
    wid = lax.axis_index("s") * NC + lax.axis_index("c")
    t0 = wid * TPW
    pltpu.sync_copy(pos_hbm.at[0, pl.ds(t0, TPW)], pidx)
    pltpu.async_copy(ys_hbm.at[pidx], buf, sem).wait()
    pltpu.sync_copy(buf, out_hbm.at[pl.ds(t0, TPW), :])


def kernel(x, Wg, bg, We, be):
    dispatch = pl.kernel(
        _dispatch_body,
        out_type=[
            jax.ShapeDtypeStruct((NP, D), jnp.float32),
            jax.ShapeDtypeStruct((NP, 128), jnp.float32),
        ],
        mesh=_sc_mesh(),
        scratch_types=[
            pltpu.VMEM((TPW,), jnp.int32),
            pltpu.VMEM((TPW, D), jnp.float32),
            pltpu.VMEM((TPW,), jnp.float32),
            pltpu.VMEM((TPW,), jnp.float32),
            pltpu.VMEM((TPW, 128), jnp.float32),
            pltpu.SemaphoreType.DMA,
        ],
    )
    combine = pl.kernel(
        _combine_body,
        out_type=jax.ShapeDtypeStruct((T, D), jnp.float32),
        mesh=_sc_mesh(),
        scratch_types=[
            pltpu.VMEM((TPW,), jnp.int32),
            pltpu.VMEM((TPW, D), jnp.float32),
            pltpu.SemaphoreType.DMA,
        ],
    )
    posT, gvT, blk8 = _router(x, Wg, bg)
    xs, gs = dispatch(x, posT, gvT)
    ys = _gemm(blk8, xs, We, be, gs)
    return combine(ys, posT)
